# baseline (speedup 1.0000x reference)
import functools

import jax
import jax.numpy as jnp
from jax import lax
from jax.experimental import pallas as pl
from jax.experimental.pallas import tpu as pltpu

_VMEM_LIMIT = 40 * 1024 * 1024
_EXT = 16


def _stage_core(xt, scale, shift, dww_ref, pww_ref, mm1_ref, mp1_ref,
                y_ref, xp_ref, *, h, w, k, p):
    c_in = xt.shape[1]
    hw = h * w
    if scale is not None:
        xt = xt * scale + shift
    xt = jnp.maximum(xt, 0.0)

    nb = xp_ref.shape[0] - hw - 64
    xp_ref[0:64, :] = jnp.zeros((64, c_in), jnp.float32)
    xp_ref[64 + hw:, :] = jnp.zeros((nb, c_in), jnp.float32)
    xp_ref[64:64 + hw, :] = xt

    dww = dww_ref[...]
    te = hw + 72 - w
    ts = []
    for kw in range(k):
        acc_t = None
        for kh in range(k):
            term = xp_ref[w * kh:w * kh + te, :] * dww[kh, kw, :]
            acc_t = term if acc_t is None else acc_t + term
        ts.append(acc_t)
    o = 64 - w - 1
    acc = (ts[1][o + 1:o + 1 + hw, :]
           + mm1_ref[...] * ts[0][o:o + hw, :]
           + mp1_ref[...] * ts[2][o + 2:o + 2 + hw, :])

    flat = acc.astype(jnp.bfloat16)
    y2d = jnp.dot(flat, pww_ref[...], preferred_element_type=jnp.float32)

    hw = h * w
    y_ref[0, 0:hw, :] = y2d.astype(y_ref.dtype)
    s = jnp.sum(y2d, axis=0, keepdims=True)
    sq = jnp.sum(y2d * y2d, axis=0, keepdims=True)
    y_ref[0, hw:hw + 1, :] = s.astype(y_ref.dtype)
    y_ref[0, hw + 1:hw + 2, :] = sq.astype(y_ref.dtype)


def _stage1_kernel(x_ref, eye_ref, dww_ref, pww_ref, mm1_ref, mp1_ref,
                   y_ref, xp_ref, *, h, w, k, p):
    xt = jax.lax.dot_general(
        x_ref[0].astype(jnp.bfloat16), eye_ref[...],
        dimension_numbers=(((0,), (0,)), ((), ())),
        preferred_element_type=jnp.float32)
    _stage_core(xt, None, None, dww_ref, pww_ref, mm1_ref, mp1_ref,
                y_ref, xp_ref, h=h, w=w, k=k, p=p)


def _stage2_kernel(x_ref, scale_ref, shift_ref, dww_ref, pww_ref,
                   mm1_ref, mp1_ref, y_ref, xp_ref, *, h, w, k, p):
    xt = x_ref[0, 0:h * w, :].astype(jnp.float32)
    _stage_core(xt, scale_ref[...], shift_ref[...], dww_ref, pww_ref,
                mm1_ref, mp1_ref, y_ref, xp_ref, h=h, w=w, k=k, p=p)


def _bn_transpose_kernel(y_ref, scale_ref, shift_ref, eye_ref, o_ref):
    hw = o_ref.shape[2]
    y = y_ref[0, 0:hw, :].astype(jnp.float32) * scale_ref[...] + shift_ref[...]
    yt = jax.lax.dot_general(
        eye_ref[...], y.astype(jnp.bfloat16),
        dimension_numbers=(((1,), (1,)), ((), ())),
        preferred_element_type=jnp.float32)
    o_ref[0] = yt.astype(o_ref.dtype)


def _finalize_stats(ye, hw, count, eps):
    s = jnp.sum(ye[:, hw, :].astype(jnp.float32), axis=0)
    sq = jnp.sum(ye[:, hw + 1, :].astype(jnp.float32), axis=0)
    mean = s / count
    var = jnp.maximum(sq / count - mean * mean, 0.0)
    scale = lax.rsqrt(var + eps)
    shift = -mean * scale
    return scale.reshape(1, -1), shift.reshape(1, -1)


def kernel(x, dw1, pw1, dw2, pw2, *, eps=1e-5):
    n, c_in, h, w = x.shape
    k = dw1.shape[0]
    c_out = pw2.shape[1]
    p = 1
    hw = h * w
    hws = hw + _EXT
    hp, wp = h + 2 * p, w + 2 * p

    pw1b = pw1.astype(jnp.bfloat16)
    pw2b = pw2.astype(jnp.bfloat16)
    eye_in = jnp.eye(c_in, dtype=jnp.bfloat16)
    eye_out = jnp.eye(c_out, dtype=jnp.bfloat16)
    x3 = x.reshape(n, c_in, hw)
    col = jnp.arange(hw, dtype=jnp.int32) % w
    mm1 = jnp.broadcast_to(
        jnp.where(col == 0, 0.0, 1.0).astype(jnp.float32)[:, None], (hw, c_in))
    mp1 = jnp.broadcast_to(
        jnp.where(col == w - 1, 0.0, 1.0).astype(jnp.float32)[:, None],
        (hw, c_in))

    y1 = pl.pallas_call(
        functools.partial(_stage1_kernel, h=h, w=w, k=k, p=p),
        out_shape=jax.ShapeDtypeStruct((n, hws, c_in), jnp.bfloat16),
        grid=(n,),
        in_specs=[pl.BlockSpec((1, c_in, hw), lambda i: (i, 0, 0)),
                  pl.BlockSpec((c_in, c_in), lambda i: (0, 0)),
                  pl.BlockSpec((k, k, c_in), lambda i: (0, 0, 0)),
                  pl.BlockSpec((c_in, c_in), lambda i: (0, 0)),
                  pl.BlockSpec((hw, c_in), lambda i: (0, 0)),
                  pl.BlockSpec((hw, c_in), lambda i: (0, 0))],
        out_specs=pl.BlockSpec((1, hws, c_in), lambda i: (i, 0, 0)),
        scratch_shapes=[pltpu.VMEM((hw + w + 72, c_in), jnp.float32)],
        compiler_params=pltpu.CompilerParams(
            dimension_semantics=("parallel",),
            vmem_limit_bytes=_VMEM_LIMIT),
    )(x3, eye_in, dw1, pw1b, mm1, mp1)
    scale1, shift1 = _finalize_stats(y1, hw, n * hw, eps)

    y2 = pl.pallas_call(
        functools.partial(_stage2_kernel, h=h, w=w, k=k, p=p),
        out_shape=jax.ShapeDtypeStruct((n, hws, c_out), jnp.bfloat16),
        grid=(n,),
        in_specs=[pl.BlockSpec((1, hws, c_in), lambda i: (i, 0, 0)),
                  pl.BlockSpec((1, c_in), lambda i: (0, 0)),
                  pl.BlockSpec((1, c_in), lambda i: (0, 0)),
                  pl.BlockSpec((k, k, c_in), lambda i: (0, 0, 0)),
                  pl.BlockSpec((c_in, c_out), lambda i: (0, 0)),
                  pl.BlockSpec((hw, c_in), lambda i: (0, 0)),
                  pl.BlockSpec((hw, c_in), lambda i: (0, 0))],
        out_specs=pl.BlockSpec((1, hws, c_out), lambda i: (i, 0, 0)),
        scratch_shapes=[pltpu.VMEM((hw + w + 72, c_in), jnp.float32)],
        compiler_params=pltpu.CompilerParams(
            dimension_semantics=("parallel",),
            vmem_limit_bytes=_VMEM_LIMIT),
    )(y1, scale1, shift1, dw2, pw2b, mm1, mp1)
    scale2, shift2 = _finalize_stats(y2, hw, n * hw, eps)

    out = pl.pallas_call(
        _bn_transpose_kernel,
        out_shape=jax.ShapeDtypeStruct((n, c_out, hw), jnp.bfloat16),
        grid=(n,),
        in_specs=[pl.BlockSpec((1, hws, c_out), lambda i: (i, 0, 0)),
                  pl.BlockSpec((1, c_out), lambda i: (0, 0)),
                  pl.BlockSpec((1, c_out), lambda i: (0, 0)),
                  pl.BlockSpec((c_out, c_out), lambda i: (0, 0))],
        out_specs=pl.BlockSpec((1, c_out, hw), lambda i: (i, 0, 0)),
        compiler_params=pltpu.CompilerParams(
            dimension_semantics=("parallel",),
            vmem_limit_bytes=_VMEM_LIMIT),
    )(y2, scale2, shift2, eye_out)
    return out.astype(jnp.float32).reshape(n, c_out, h, w)


# speedup vs baseline: 1.0736x; 1.0736x over previous
import functools

import jax
import jax.numpy as jnp
from jax import lax
from jax.experimental import pallas as pl
from jax.experimental.pallas import tpu as pltpu

_VMEM_LIMIT = 40 * 1024 * 1024


def _stage_core(xt, scale, shift, dww_ref, pww_ref, y_ref, stats_ref, xp_ref,
                b, *, h, w, k, p):
    c_in = xt.shape[1]
    hp, wp = h + 2 * p, w + 2 * p
    if scale is not None:
        xt = xt * scale + shift
    xt = jnp.maximum(xt, 0.0)

    zrow = jnp.zeros((p, wp, c_in), jnp.float32)
    xp_ref[0:p, :, :] = zrow
    xp_ref[h + p:hp, :, :] = zrow
    zcol = jnp.zeros((h, p, c_in), jnp.float32)
    xp_ref[p:p + h, 0:p, :] = zcol
    xp_ref[p:p + h, w + p:wp, :] = zcol
    xp_ref[p:p + h, p:p + w, :] = xt.reshape(h, w, c_in)

    dww = dww_ref[...]
    acc = None
    for kh in range(k):
        for kw in range(k):
            term = xp_ref[kh:kh + h, kw:kw + w, :] * dww[kh, kw, :]
            acc = term if acc is None else acc + term

    flat = acc.reshape(h * w, c_in).astype(jnp.bfloat16)
    y2d = jnp.dot(flat, pww_ref[...], preferred_element_type=jnp.float32)

    stats_ref[b, 0:1, :] = jnp.sum(y2d, axis=0, keepdims=True)
    stats_ref[b, 1:2, :] = jnp.sum(y2d * y2d, axis=0, keepdims=True)
    y_ref[b] = y2d.astype(y_ref.dtype)


def _stage1_kernel(x_ref, eye_ref, dww_ref, pww_ref, y_ref, stats_ref, xp_ref,
                   *, h, w, k, p, nb):
    for b in range(nb):
        xt = jax.lax.dot_general(
            x_ref[b].astype(jnp.bfloat16), eye_ref[...],
            dimension_numbers=(((0,), (0,)), ((), ())),
            preferred_element_type=jnp.float32)
        _stage_core(xt, None, None, dww_ref, pww_ref,
                    y_ref, stats_ref, xp_ref, b, h=h, w=w, k=k, p=p)


def _stage2_kernel(x_ref, scale_ref, shift_ref, dww_ref, pww_ref,
                   y_ref, stats_ref, xp_ref, *, h, w, k, p, nb):
    for b in range(nb):
        xt = x_ref[b].astype(jnp.float32)
        _stage_core(xt, scale_ref[...], shift_ref[...], dww_ref, pww_ref,
                    y_ref, stats_ref, xp_ref, b, h=h, w=w, k=k, p=p)


def _bn_transpose_kernel(y_ref, scale_ref, shift_ref, eye_ref, o_ref, *, nb):
    for b in range(nb):
        y = y_ref[b].astype(jnp.float32) * scale_ref[...] + shift_ref[...]
        yt = jax.lax.dot_general(
            eye_ref[...], y.astype(jnp.bfloat16),
            dimension_numbers=(((1,), (1,)), ((), ())),
            preferred_element_type=jnp.float32)
        o_ref[b] = yt.astype(o_ref.dtype)


def _finalize_stats(stats, count, eps):
    s = jnp.sum(stats[:, 0, :], axis=0)
    sq = jnp.sum(stats[:, 1, :], axis=0)
    mean = s / count
    var = jnp.maximum(sq / count - mean * mean, 0.0)
    scale = lax.rsqrt(var + eps)
    shift = -mean * scale
    return scale.reshape(1, -1), shift.reshape(1, -1)


def kernel(x, dw1, pw1, dw2, pw2, *, eps=1e-5):
    n, c_in, h, w = x.shape
    k = dw1.shape[0]
    c_out = pw2.shape[1]
    p = 1
    hw = h * w
    hp, wp = h + 2 * p, w + 2 * p

    nb = 2 if n % 2 == 0 else 1
    pw1b = pw1.astype(jnp.bfloat16)
    pw2b = pw2.astype(jnp.bfloat16)
    eye_in = jnp.eye(c_in, dtype=jnp.bfloat16)
    eye_out = jnp.eye(c_out, dtype=jnp.bfloat16)
    x3 = x.reshape(n, c_in, hw)

    y1, stats1 = pl.pallas_call(
        functools.partial(_stage1_kernel, h=h, w=w, k=k, p=p, nb=nb),
        out_shape=(jax.ShapeDtypeStruct((n, hw, c_in), jnp.bfloat16),
                   jax.ShapeDtypeStruct((n, 2, c_in), jnp.float32)),
        grid=(n // nb,),
        in_specs=[pl.BlockSpec((nb, c_in, hw), lambda i: (i, 0, 0)),
                  pl.BlockSpec((c_in, c_in), lambda i: (0, 0)),
                  pl.BlockSpec((k, k, c_in), lambda i: (0, 0, 0)),
                  pl.BlockSpec((c_in, c_in), lambda i: (0, 0))],
        out_specs=(pl.BlockSpec((nb, hw, c_in), lambda i: (i, 0, 0)),
                   pl.BlockSpec((nb, 2, c_in), lambda i: (i, 0, 0))),
        scratch_shapes=[pltpu.VMEM((hp, wp, c_in), jnp.float32)],
        compiler_params=pltpu.CompilerParams(
            dimension_semantics=("parallel",),
            vmem_limit_bytes=_VMEM_LIMIT),
    )(x3, eye_in, dw1, pw1b)
    scale1, shift1 = _finalize_stats(stats1, n * hw, eps)

    y2, stats2 = pl.pallas_call(
        functools.partial(_stage2_kernel, h=h, w=w, k=k, p=p, nb=nb),
        out_shape=(jax.ShapeDtypeStruct((n, hw, c_out), jnp.bfloat16),
                   jax.ShapeDtypeStruct((n, 2, c_out), jnp.float32)),
        grid=(n // nb,),
        in_specs=[pl.BlockSpec((nb, hw, c_in), lambda i: (i, 0, 0)),
                  pl.BlockSpec((1, c_in), lambda i: (0, 0)),
                  pl.BlockSpec((1, c_in), lambda i: (0, 0)),
                  pl.BlockSpec((k, k, c_in), lambda i: (0, 0, 0)),
                  pl.BlockSpec((c_in, c_out), lambda i: (0, 0))],
        out_specs=(pl.BlockSpec((nb, hw, c_out), lambda i: (i, 0, 0)),
                   pl.BlockSpec((nb, 2, c_out), lambda i: (i, 0, 0))),
        scratch_shapes=[pltpu.VMEM((hp, wp, c_in), jnp.float32)],
        compiler_params=pltpu.CompilerParams(
            dimension_semantics=("parallel",),
            vmem_limit_bytes=_VMEM_LIMIT),
    )(y1, scale1, shift1, dw2, pw2b)
    scale2, shift2 = _finalize_stats(stats2, n * hw, eps)

    out = pl.pallas_call(
        functools.partial(_bn_transpose_kernel, nb=nb),
        out_shape=jax.ShapeDtypeStruct((n, c_out, hw), jnp.bfloat16),
        grid=(n // nb,),
        in_specs=[pl.BlockSpec((nb, hw, c_out), lambda i: (i, 0, 0)),
                  pl.BlockSpec((1, c_out), lambda i: (0, 0)),
                  pl.BlockSpec((1, c_out), lambda i: (0, 0)),
                  pl.BlockSpec((c_out, c_out), lambda i: (0, 0))],
        out_specs=pl.BlockSpec((nb, c_out, hw), lambda i: (i, 0, 0)),
        compiler_params=pltpu.CompilerParams(
            dimension_semantics=("parallel",),
            vmem_limit_bytes=_VMEM_LIMIT),
    )(y2, scale2, shift2, eye_out)
    return out.astype(jnp.float32).reshape(n, c_out, h, w)


# speedup vs baseline: 1.0912x; 1.0164x over previous
import functools

import jax
import jax.numpy as jnp
from jax import lax
from jax.experimental import pallas as pl
from jax.experimental.pallas import tpu as pltpu

_VMEM_LIMIT = 40 * 1024 * 1024


def _stage_core(xt, scale, shift, dww_ref, pww_ref, y_ref, stats_ref, xp_ref,
                b, *, h, w, k, p):
    c_in = xt.shape[1]
    hp, wp = h + 2 * p, w + 2 * p
    if scale is not None:
        xt = xt * scale + shift
    xt = jnp.maximum(xt, 0.0)

    zrow = jnp.zeros((p, wp, c_in), jnp.float32)
    xp_ref[0:p, :, :] = zrow
    xp_ref[h + p:hp, :, :] = zrow
    zcol = jnp.zeros((h, p, c_in), jnp.float32)
    xp_ref[p:p + h, 0:p, :] = zcol
    xp_ref[p:p + h, w + p:wp, :] = zcol
    xp_ref[p:p + h, p:p + w, :] = xt.reshape(h, w, c_in)

    dww = dww_ref[...]
    acc = None
    for kh in range(k):
        for kw in range(k):
            term = xp_ref[kh:kh + h, kw:kw + w, :] * dww[kh, kw, :]
            acc = term if acc is None else acc + term

    flat = acc.reshape(h * w, c_in).astype(jnp.bfloat16)
    y2d = jnp.dot(flat, pww_ref[...], preferred_element_type=jnp.float32)

    stats_ref[b, 0:1, :] = jnp.sum(y2d, axis=0, keepdims=True)
    stats_ref[b, 1:2, :] = jnp.sum(y2d * y2d, axis=0, keepdims=True)
    y_ref[b] = y2d.astype(y_ref.dtype)


def _stage1_kernel(x_ref, eye_ref, dww_ref, pww_ref, y_ref, stats_ref, xp_ref,
                   *, h, w, k, p, nb):
    for b in range(nb):
        xt = jax.lax.dot_general(
            x_ref[b].astype(jnp.bfloat16), eye_ref[...],
            dimension_numbers=(((0,), (0,)), ((), ())),
            preferred_element_type=jnp.float32)
        _stage_core(xt, None, None, dww_ref, pww_ref,
                    y_ref, stats_ref, xp_ref, b, h=h, w=w, k=k, p=p)


def _stage2_kernel(x_ref, scale_ref, shift_ref, dww_ref, pww_ref,
                   y_ref, stats_ref, xp_ref, *, h, w, k, p, nb):
    for b in range(nb):
        xt = x_ref[b].astype(jnp.float32)
        _stage_core(xt, scale_ref[...], shift_ref[...], dww_ref, pww_ref,
                    y_ref, stats_ref, xp_ref, b, h=h, w=w, k=k, p=p)


def _bn_transpose_kernel(y_ref, scale_ref, shift_ref, eye_ref, o_ref, *, nb):
    for b in range(nb):
        y = y_ref[b].astype(jnp.float32) * scale_ref[...] + shift_ref[...]
        yt = jax.lax.dot_general(
            eye_ref[...], y.astype(jnp.bfloat16),
            dimension_numbers=(((1,), (1,)), ((), ())),
            preferred_element_type=jnp.float32)
        o_ref[b] = yt.astype(o_ref.dtype)


def _finalize_stats(stats, count, eps):
    s = jnp.sum(stats[:, 0, :], axis=0)
    sq = jnp.sum(stats[:, 1, :], axis=0)
    mean = s / count
    var = jnp.maximum(sq / count - mean * mean, 0.0)
    scale = lax.rsqrt(var + eps)
    shift = -mean * scale
    return scale.reshape(1, -1), shift.reshape(1, -1)


def kernel(x, dw1, pw1, dw2, pw2, *, eps=1e-5):
    n, c_in, h, w = x.shape
    k = dw1.shape[0]
    c_out = pw2.shape[1]
    p = 1
    hw = h * w
    hp, wp = h + 2 * p, w + 2 * p

    nb = 4 if n % 4 == 0 else (2 if n % 2 == 0 else 1)
    pw1b = pw1.astype(jnp.bfloat16)
    pw2b = pw2.astype(jnp.bfloat16)
    eye_in = jnp.eye(c_in, dtype=jnp.bfloat16)
    eye_out = jnp.eye(c_out, dtype=jnp.bfloat16)
    x3 = x.reshape(n, c_in, hw)

    y1, stats1 = pl.pallas_call(
        functools.partial(_stage1_kernel, h=h, w=w, k=k, p=p, nb=nb),
        out_shape=(jax.ShapeDtypeStruct((n, hw, c_in), jnp.bfloat16),
                   jax.ShapeDtypeStruct((n, 2, c_in), jnp.float32)),
        grid=(n // nb,),
        in_specs=[pl.BlockSpec((nb, c_in, hw), lambda i: (i, 0, 0)),
                  pl.BlockSpec((c_in, c_in), lambda i: (0, 0)),
                  pl.BlockSpec((k, k, c_in), lambda i: (0, 0, 0)),
                  pl.BlockSpec((c_in, c_in), lambda i: (0, 0))],
        out_specs=(pl.BlockSpec((nb, hw, c_in), lambda i: (i, 0, 0)),
                   pl.BlockSpec((nb, 2, c_in), lambda i: (i, 0, 0))),
        scratch_shapes=[pltpu.VMEM((hp, wp, c_in), jnp.float32)],
        compiler_params=pltpu.CompilerParams(
            dimension_semantics=("parallel",),
            vmem_limit_bytes=_VMEM_LIMIT),
    )(x3, eye_in, dw1, pw1b)
    scale1, shift1 = _finalize_stats(stats1, n * hw, eps)

    y2, stats2 = pl.pallas_call(
        functools.partial(_stage2_kernel, h=h, w=w, k=k, p=p, nb=nb),
        out_shape=(jax.ShapeDtypeStruct((n, hw, c_out), jnp.bfloat16),
                   jax.ShapeDtypeStruct((n, 2, c_out), jnp.float32)),
        grid=(n // nb,),
        in_specs=[pl.BlockSpec((nb, hw, c_in), lambda i: (i, 0, 0)),
                  pl.BlockSpec((1, c_in), lambda i: (0, 0)),
                  pl.BlockSpec((1, c_in), lambda i: (0, 0)),
                  pl.BlockSpec((k, k, c_in), lambda i: (0, 0, 0)),
                  pl.BlockSpec((c_in, c_out), lambda i: (0, 0))],
        out_specs=(pl.BlockSpec((nb, hw, c_out), lambda i: (i, 0, 0)),
                   pl.BlockSpec((nb, 2, c_out), lambda i: (i, 0, 0))),
        scratch_shapes=[pltpu.VMEM((hp, wp, c_in), jnp.float32)],
        compiler_params=pltpu.CompilerParams(
            dimension_semantics=("parallel",),
            vmem_limit_bytes=_VMEM_LIMIT),
    )(y1, scale1, shift1, dw2, pw2b)
    scale2, shift2 = _finalize_stats(stats2, n * hw, eps)

    out = pl.pallas_call(
        functools.partial(_bn_transpose_kernel, nb=nb),
        out_shape=jax.ShapeDtypeStruct((n, c_out, hw), jnp.bfloat16),
        grid=(n // nb,),
        in_specs=[pl.BlockSpec((nb, hw, c_out), lambda i: (i, 0, 0)),
                  pl.BlockSpec((1, c_out), lambda i: (0, 0)),
                  pl.BlockSpec((1, c_out), lambda i: (0, 0)),
                  pl.BlockSpec((c_out, c_out), lambda i: (0, 0))],
        out_specs=pl.BlockSpec((nb, c_out, hw), lambda i: (i, 0, 0)),
        compiler_params=pltpu.CompilerParams(
            dimension_semantics=("parallel",),
            vmem_limit_bytes=_VMEM_LIMIT),
    )(y2, scale2, shift2, eye_out)
    return out.astype(jnp.float32).reshape(n, c_out, h, w)
